# baseline (speedup 1.0000x reference)
import functools
import jax
import jax.numpy as jnp
from jax import lax
from jax.experimental import pallas as pl
from jax.experimental.pallas import tpu as pltpu
from jax.experimental.pallas import tpu_sc as plsc

B, NU, NE, D = 64, 2048, 16, 256
G = 4
NC, NS = 2, 16
BPW = B // (NC * NS)
CHUNKS = NU // 16


def _tc_kernel(user_h_ref, ego_h_ref, user_mask_ref, ego_mask_ref,
               hgs_ref):
    for g in range(G):
        mu = (user_mask_ref[g] > 0).astype(jnp.bfloat16)
        uh = user_h_ref[g].astype(jnp.bfloat16)
        num_u = jnp.dot(mu, uh, preferred_element_type=jnp.float32)
        cnt_u = jnp.maximum(
            jnp.sum(mu.astype(jnp.float32), axis=1, keepdims=True), 1.0)

        me = (ego_mask_ref[g] > 0).astype(jnp.float32)
        eh = ego_h_ref[g]
        num_e = jnp.dot(me, eh, preferred_element_type=jnp.float32)
        cnt_e = jnp.maximum(jnp.sum(me, axis=1, keepdims=True), 1.0)

        hgs_ref[g * NE:(g + 1) * NE, :] = num_u / cnt_u + num_e / cnt_e


def _sc_adj_kernel(neigh_hbm, adj_hbm, nv, adjv):
    wid = lax.axis_index("s") * NC + lax.axis_index("c")
    lane = lax.iota(jnp.int32, 16)
    for i in range(BPW):
        b = wid * BPW + i
        pltpu.sync_copy(neigh_hbm.at[b], nv)

        def chunk_body(c, rows):
            bm = jnp.zeros((16,), jnp.int32)
            for j in range(NE):
                a = nv[j, pl.ds(c * 16, 16)]
                bm = bm | jnp.where(a > 0, jnp.int32(1 << j), jnp.int32(0))
            new_rows = []
            for j in range(NE):
                hit = ((bm >> j) & 1) != 0
                new_rows.append(rows[j] | jnp.where(hit, bm, jnp.int32(0)))
            return tuple(new_rows)

        rows = lax.fori_loop(
            0, CHUNKS, chunk_body,
            tuple(jnp.zeros((16,), jnp.int32) for _ in range(NE)))

        for j in range(NE):
            r = rows[j]
            for s in (8, 4, 2, 1):
                r = r | r.at[lane ^ s].get(mode="promise_in_bounds")
            adj_row = (r >> lane) & 1
            adjv[j, :] = jnp.where(lane == j, jnp.int32(0), adj_row)
        pltpu.sync_copy(adjv, adj_hbm.at[b])


@functools.partial(
    pl.kernel,
    out_type=jax.ShapeDtypeStruct((B, NE, NE), jnp.int32),
    mesh=plsc.VectorSubcoreMesh(core_axis_name="c", subcore_axis_name="s"),
    scratch_types=[
        pltpu.VMEM((NE, NU), jnp.int32),
        pltpu.VMEM((NE, NE), jnp.int32),
    ],
)
def _sc_adj(neigh_hbm, adj_hbm, nv, adjv):
    _sc_adj_kernel(neigh_hbm, adj_hbm, nv, adjv)


def kernel(user_h, ego_h, user_mask, ego_mask, neigh):
    adj_i = _sc_adj(neigh)
    hgs = pl.pallas_call(
        _tc_kernel,
        grid=(B // G,),
        in_specs=[
            pl.BlockSpec((G, NU, D), lambda b: (b, 0, 0)),
            pl.BlockSpec((G, NE, D), lambda b: (b, 0, 0)),
            pl.BlockSpec((G, NE, NU), lambda b: (b, 0, 0)),
            pl.BlockSpec((G, NE, NE), lambda b: (b, 0, 0)),
        ],
        out_specs=pl.BlockSpec((G * NE, D), lambda b: (b, 0)),
        out_shape=jax.ShapeDtypeStruct((B * NE, D), jnp.float32),
    )(user_h, ego_h, user_mask, ego_mask)
    return hgs, adj_i.astype(bool)


# speedup vs baseline: 1.3027x; 1.3027x over previous
import jax
import jax.numpy as jnp
from jax import lax
from jax.experimental import pallas as pl
from jax.experimental.pallas import tpu as pltpu

B, NU, NE, D = 64, 2048, 16, 256
NBUF = 4


def _ring_kernel(uh_any, um_any, nf_any, eh_ref, em_ref,
                 hgs_ref, adj_ref, uh_buf, um_buf, nf_buf, sems):
    def copy_in(b, slot):
        pltpu.make_async_copy(uh_any.at[b], uh_buf.at[slot],
                              sems.at[0, slot]).start()
        pltpu.make_async_copy(um_any.at[b], um_buf.at[slot],
                              sems.at[1, slot]).start()
        pltpu.make_async_copy(nf_any.at[b], nf_buf.at[slot],
                              sems.at[2, slot]).start()

    for i in range(NBUF):
        copy_in(i, i)

    row = lax.broadcasted_iota(jnp.int32, (NE, NE), 0)
    col = lax.broadcasted_iota(jnp.int32, (NE, NE), 1)

    def body(b, carry):
        slot = lax.rem(b, NBUF)
        pltpu.make_async_copy(uh_any.at[b], uh_buf.at[slot],
                              sems.at[0, slot]).wait()
        pltpu.make_async_copy(um_any.at[b], um_buf.at[slot],
                              sems.at[1, slot]).wait()
        pltpu.make_async_copy(nf_any.at[b], nf_buf.at[slot],
                              sems.at[2, slot]).wait()

        mu = (um_buf[slot] > 0).astype(jnp.bfloat16)
        uh = uh_buf[slot].astype(jnp.bfloat16)
        num_u = jnp.dot(mu, uh, preferred_element_type=jnp.float32)
        cnt_u = jnp.maximum(
            jnp.sum(mu.astype(jnp.float32), axis=1, keepdims=True), 1.0)

        me = (em_ref[b] > 0).astype(jnp.float32)
        eh = eh_ref[b]
        num_e = jnp.dot(me, eh, preferred_element_type=jnp.float32)
        cnt_e = jnp.maximum(jnp.sum(me, axis=1, keepdims=True), 1.0)

        hgs_ref[pl.ds(b * NE, NE), :] = num_u / cnt_u + num_e / cnt_e

        nf = (nf_buf[slot] > 0).astype(jnp.bfloat16)
        ov = jnp.dot(nf, nf.T, preferred_element_type=jnp.float32)
        adj_ref[b] = ((ov > 0.0) & (row != col)).astype(jnp.int32)

        @pl.when(b + NBUF < B)
        def _():
            copy_in(b + NBUF, slot)

        return carry

    lax.fori_loop(0, B, body, 0)


def kernel(user_h, ego_h, user_mask, ego_mask, neigh):
    hgs, adj_i = pl.pallas_call(
        _ring_kernel,
        in_specs=[
            pl.BlockSpec(memory_space=pl.ANY),
            pl.BlockSpec(memory_space=pl.ANY),
            pl.BlockSpec(memory_space=pl.ANY),
            pl.BlockSpec((B, NE, D), lambda: (0, 0, 0)),
            pl.BlockSpec((B, NE, NE), lambda: (0, 0, 0)),
        ],
        out_specs=[
            pl.BlockSpec((B * NE, D), lambda: (0, 0)),
            pl.BlockSpec((B, NE, NE), lambda: (0, 0, 0)),
        ],
        out_shape=[
            jax.ShapeDtypeStruct((B * NE, D), jnp.float32),
            jax.ShapeDtypeStruct((B, NE, NE), jnp.int32),
        ],
        scratch_shapes=[
            pltpu.VMEM((NBUF, NU, D), jnp.float32),
            pltpu.VMEM((NBUF, NE, NU), jnp.int32),
            pltpu.VMEM((NBUF, NE, NU), jnp.int32),
            pltpu.SemaphoreType.DMA((3, NBUF)),
        ],
    )(user_h, user_mask, neigh, ego_h, ego_mask)
    return hgs, adj_i.astype(bool)
